# initial kernel scaffold (unmeasured)
import jax
import jax.numpy as jnp
from jax import lax
from jax.experimental import pallas as pl
from jax.experimental.pallas import tpu as pltpu

N_DEV = 4
SQ_SHARD = 256
HQ_SHARD = 8
DH = 128
SKV = 4096
D_MODEL = 1024
BLK = 64
SCALE = 0.08838834764831843
NEG_INF = -1e9


def kernel(x, Wq, K_ext, V_ext, Wo):
    def body(x_ref, wq_ref, k_hbm, v_hbm, wo_ref, out_ref,
             xg, pbuf, rs_buf, k_bf, v_bf, kv_stage,
             kv_sem, ag_send, ag_recv, rs_send, rs_recv):
        my = lax.axis_index("i")
        right = lax.rem(my + 1, N_DEV)
        left = lax.rem(my + N_DEV - 1, N_DEV)

        barrier = pltpu.get_barrier_semaphore()
        for nbr in (left, right):
            pl.semaphore_signal(barrier, inc=1, device_id=(nbr,),
                                device_id_type=pl.DeviceIdType.MESH)
        pl.semaphore_wait(barrier, 2)

        for hbm, dst in ((k_hbm, k_bf), (v_hbm, v_bf)):
            for part in range(2):
                cp = pltpu.make_async_copy(
                    hbm.at[0, :, pl.ds(my * HQ_SHARD + part * 4, 4), :],
                    kv_stage, kv_sem.at[0])
                cp.start()
                cp.wait()
                dst[:, part * 4:(part + 1) * 4, :] = (
                    kv_stage[:, :, :].astype(jnp.bfloat16))

        wq_bf = wq_ref[:, :].astype(jnp.bfloat16)
        wo_bf = wo_ref[:, :].astype(jnp.bfloat16)

        xg[my] = x_ref[0].astype(jnp.bfloat16)

        def compute_chunk(c):
            origin = lax.rem(my - c + 2 * N_DEV, N_DEV)
            x_c = xg[origin]
            q = lax.dot(x_c, wq_bf, preferred_element_type=jnp.float32)
            q_bf = q.astype(jnp.bfloat16)

            r = lax.broadcasted_iota(jnp.int32, (SQ_SHARD, SKV), 0)
            ccol = lax.broadcasted_iota(jnp.int32, (SQ_SHARD, SKV), 1)
            qb = r // BLK + origin * (SQ_SHARD // BLK)
            kb = ccol // BLK
            mask = (qb == kb) | (kb == 0) | (lax.rem(qb + kb, 3) == 0)

            ctxs = []
            for h in range(HQ_SHARD):
                q_h = q_bf[:, h * DH:(h + 1) * DH]
                s = lax.dot_general(
                    q_h, k_bf[:, h, :], (((1,), (1,)), ((), ())),
                    preferred_element_type=jnp.float32) * SCALE
                s = jnp.where(mask, s, NEG_INF)
                mx = jnp.max(s, axis=1, keepdims=True)
                w = jnp.exp(s - mx)
                w = (w / jnp.sum(w, axis=1, keepdims=True)).astype(jnp.bfloat16)
                ctxs.append(lax.dot_general(
                    w, v_bf[:, h, :], (((1,), (0,)), ((), ())),
                    preferred_element_type=jnp.float32))
            ctx = jnp.concatenate(ctxs, axis=1).astype(jnp.bfloat16)
            pbuf[origin] = lax.dot(ctx, wo_bf,
                                   preferred_element_type=jnp.float32)

        for h in range(N_DEV - 1):
            slot = lax.rem(my - h + 2 * N_DEV, N_DEV)
            rdma = pltpu.make_async_remote_copy(
                src_ref=xg.at[slot], dst_ref=xg.at[slot],
                send_sem=ag_send.at[h], recv_sem=ag_recv.at[h],
                device_id=(right,), device_id_type=pl.DeviceIdType.MESH)
            rdma.start()
            compute_chunk(h)
            rdma.wait()
        compute_chunk(N_DEV - 1)

        for s in range(N_DEV - 1):
            c_send = lax.rem(my - 1 - s + 2 * N_DEV, N_DEV)
            rdma = pltpu.make_async_remote_copy(
                src_ref=pbuf.at[c_send], dst_ref=rs_buf.at[s],
                send_sem=rs_send.at[s], recv_sem=rs_recv.at[s],
                device_id=(right,), device_id_type=pl.DeviceIdType.MESH)
            rdma.start()
            rdma.wait()
            c_recv = lax.rem(my - 2 - s + 2 * N_DEV, N_DEV)
            pbuf[c_recv] = pbuf[c_recv] + rs_buf[s]

        out_ref[0] = pbuf[my]

    return pl.pallas_call(
        body,
        out_shape=jax.ShapeDtypeStruct((1, SQ_SHARD, D_MODEL), jnp.float32),
        in_specs=[
            pl.BlockSpec(memory_space=pltpu.VMEM),
            pl.BlockSpec(memory_space=pltpu.VMEM),
            pl.BlockSpec(memory_space=pltpu.ANY),
            pl.BlockSpec(memory_space=pltpu.ANY),
            pl.BlockSpec(memory_space=pltpu.VMEM),
        ],
        out_specs=pl.BlockSpec(memory_space=pltpu.VMEM),
        scratch_shapes=[
            pltpu.VMEM((N_DEV, SQ_SHARD, D_MODEL), jnp.bfloat16),
            pltpu.VMEM((N_DEV, SQ_SHARD, D_MODEL), jnp.float32),
            pltpu.VMEM((N_DEV - 1, SQ_SHARD, D_MODEL), jnp.float32),
            pltpu.VMEM((SKV, HQ_SHARD, DH), jnp.bfloat16),
            pltpu.VMEM((SKV, HQ_SHARD, DH), jnp.bfloat16),
            pltpu.VMEM((SKV, 4, DH), jnp.float32),
            pltpu.SemaphoreType.DMA((1,)),
            pltpu.SemaphoreType.DMA((3,)),
            pltpu.SemaphoreType.DMA((3,)),
            pltpu.SemaphoreType.DMA((3,)),
            pltpu.SemaphoreType.DMA((3,)),
        ],
        compiler_params=pltpu.CompilerParams(collective_id=0),
    )(x, Wq, K_ext, V_ext, Wo)


# baseline (device time: 173777 ns/iter reference)
import jax
import jax.numpy as jnp
from jax import lax
from jax.experimental import pallas as pl
from jax.experimental.pallas import tpu as pltpu

N_DEV = 4
SQ_SHARD = 256
HQ_SHARD = 8
DH = 128
SKV = 4096
D_MODEL = 1024
BLK = 64
SCALE = 0.08838834764831843
NEG_INF = -1e9


def kernel(x, Wq, K_ext, V_ext, Wo):
    def body(x_ref, wq_ref, k_hbm, v_hbm, wo_ref, out_ref,
             xg, pbuf, rs_buf, k_bf, v_bf, kv_stage, q_scr, wo_scr,
             kv_sem, ag_send, ag_recv, rs_send, rs_recv):
        my = lax.axis_index("i")
        right = lax.rem(my + 1, N_DEV)
        left = lax.rem(my + N_DEV - 1, N_DEV)

        barrier = pltpu.get_barrier_semaphore()
        for nbr in (left, right):
            pl.semaphore_signal(barrier, inc=1, device_id=(nbr,),
                                device_id_type=pl.DeviceIdType.MESH)
        pl.semaphore_wait(barrier, 2)

        for hbm, dst in ((k_hbm, k_bf), (v_hbm, v_bf)):
            def load_head(h, _):
                cp = pltpu.make_async_copy(
                    hbm.at[0, :, my * HQ_SHARD + h, :], kv_stage,
                    kv_sem.at[0])
                cp.start()
                cp.wait()
                dst[h] = kv_stage[:, :].astype(jnp.bfloat16)
                return 0
            lax.fori_loop(0, HQ_SHARD, load_head, 0)

        wq_bf = wq_ref[:, :].astype(jnp.bfloat16)
        for h in range(HQ_SHARD):
            wo_scr[h] = wo_ref[h * DH:(h + 1) * DH, :].astype(jnp.bfloat16)

        xg[my] = x_ref[0].astype(jnp.bfloat16)

        def compute_chunk(c):
            origin = lax.rem(my - c + 2 * N_DEV, N_DEV)
            x_c = xg[origin]
            q = lax.dot(x_c, wq_bf, preferred_element_type=jnp.float32)
            q_bf = q.astype(jnp.bfloat16)
            for h in range(HQ_SHARD):
                q_scr[h] = q_bf[:, h * DH:(h + 1) * DH]

            r = lax.broadcasted_iota(jnp.int32, (SQ_SHARD, SKV), 0)
            ccol = lax.broadcasted_iota(jnp.int32, (SQ_SHARD, SKV), 1)
            qb = r // BLK + origin * (SQ_SHARD // BLK)
            kb = ccol // BLK
            mask = (qb == kb) | (kb == 0) | (lax.rem(qb + kb, 3) == 0)

            def head_body(h, acc):
                s = lax.dot_general(
                    q_scr[h], k_bf[h], (((1,), (1,)), ((), ())),
                    preferred_element_type=jnp.float32) * SCALE
                s = jnp.where(mask, s, NEG_INF)
                mx = jnp.max(s, axis=1, keepdims=True)
                w = jnp.exp(s - mx)
                w = (w / jnp.sum(w, axis=1, keepdims=True)).astype(jnp.bfloat16)
                ctx = lax.dot_general(
                    w, v_bf[h], (((1,), (0,)), ((), ())),
                    preferred_element_type=jnp.float32)
                return acc + lax.dot(ctx.astype(jnp.bfloat16), wo_scr[h],
                                     preferred_element_type=jnp.float32)

            acc0 = jnp.zeros((SQ_SHARD, D_MODEL), jnp.float32)
            pbuf[origin] = lax.fori_loop(0, HQ_SHARD, head_body, acc0)

        for h in range(N_DEV - 1):
            slot = lax.rem(my - h + 2 * N_DEV, N_DEV)
            rdma = pltpu.make_async_remote_copy(
                src_ref=xg.at[slot], dst_ref=xg.at[slot],
                send_sem=ag_send.at[h], recv_sem=ag_recv.at[h],
                device_id=(right,), device_id_type=pl.DeviceIdType.MESH)
            rdma.start()
            compute_chunk(h)
            rdma.wait()
        compute_chunk(N_DEV - 1)

        for s in range(N_DEV - 1):
            c_send = lax.rem(my - 1 - s + 2 * N_DEV, N_DEV)
            rdma = pltpu.make_async_remote_copy(
                src_ref=pbuf.at[c_send], dst_ref=rs_buf.at[s],
                send_sem=rs_send.at[s], recv_sem=rs_recv.at[s],
                device_id=(right,), device_id_type=pl.DeviceIdType.MESH)
            rdma.start()
            rdma.wait()
            c_recv = lax.rem(my - 2 - s + 2 * N_DEV, N_DEV)
            pbuf[c_recv] = pbuf[c_recv] + rs_buf[s]

        out_ref[0] = pbuf[my]

    return pl.pallas_call(
        body,
        out_shape=jax.ShapeDtypeStruct((1, SQ_SHARD, D_MODEL), jnp.float32),
        in_specs=[
            pl.BlockSpec(memory_space=pltpu.VMEM),
            pl.BlockSpec(memory_space=pltpu.VMEM),
            pl.BlockSpec(memory_space=pl.ANY),
            pl.BlockSpec(memory_space=pl.ANY),
            pl.BlockSpec(memory_space=pltpu.VMEM),
        ],
        out_specs=pl.BlockSpec(memory_space=pltpu.VMEM),
        scratch_shapes=[
            pltpu.VMEM((N_DEV, SQ_SHARD, D_MODEL), jnp.bfloat16),
            pltpu.VMEM((N_DEV, SQ_SHARD, D_MODEL), jnp.float32),
            pltpu.VMEM((N_DEV - 1, SQ_SHARD, D_MODEL), jnp.float32),
            pltpu.VMEM((HQ_SHARD, SKV, DH), jnp.bfloat16),
            pltpu.VMEM((HQ_SHARD, SKV, DH), jnp.bfloat16),
            pltpu.VMEM((SKV, DH), jnp.float32),
            pltpu.VMEM((HQ_SHARD, SQ_SHARD, DH), jnp.bfloat16),
            pltpu.VMEM((HQ_SHARD, DH, D_MODEL), jnp.bfloat16),
            pltpu.SemaphoreType.DMA((1,)),
            pltpu.SemaphoreType.DMA((3,)),
            pltpu.SemaphoreType.DMA((3,)),
            pltpu.SemaphoreType.DMA((3,)),
            pltpu.SemaphoreType.DMA((3,)),
        ],
        compiler_params=pltpu.CompilerParams(
            collective_id=0, vmem_limit_bytes=56 * 1024 * 1024),
    )(x, Wq, K_ext, V_ext, Wo)


# device time: 149346 ns/iter; 1.1636x vs baseline; 1.1636x over previous
import jax
import jax.numpy as jnp
from jax import lax
from jax.experimental import pallas as pl
from jax.experimental.pallas import tpu as pltpu

N_DEV = 4
SQ_SHARD = 256
HQ_SHARD = 8
DH = 128
SKV = 4096
D_MODEL = 1024
BLK = 64
SCALE = 0.08838834764831843
NEG_INF = -1e9


def kernel(x, Wq, K_ext, V_ext, Wo):
    def body(x_ref, wq_ref, k_hbm, v_hbm, wo_ref, out_ref,
             xg, pbuf, rs_buf, k_bf, v_bf, kv_stage, q_scr,
             kv_sem, ag_send, ag_recv, rs_send, rs_recv):
        my = lax.axis_index("i")
        right = lax.rem(my + 1, N_DEV)
        left = lax.rem(my + N_DEV - 1, N_DEV)

        barrier = pltpu.get_barrier_semaphore()
        for nbr in (left, right):
            pl.semaphore_signal(barrier, inc=1, device_id=(nbr,),
                                device_id_type=pl.DeviceIdType.MESH)
        pl.semaphore_wait(barrier, 2)

        xg[my] = x_ref[0]
        ag0 = pltpu.make_async_remote_copy(
            src_ref=xg.at[my], dst_ref=xg.at[my],
            send_sem=ag_send.at[0], recv_sem=ag_recv.at[0],
            device_id=(right,), device_id_type=pl.DeviceIdType.MESH)
        ag0.start()

        for hbm, dst in ((k_hbm, k_bf), (v_hbm, v_bf)):
            def load_head(h, _):
                cp = pltpu.make_async_copy(
                    hbm.at[0, :, my * HQ_SHARD + h, :], kv_stage,
                    kv_sem.at[0])
                cp.start()
                cp.wait()
                dst[h] = kv_stage[:, :].astype(jnp.bfloat16)
                return 0
            lax.fori_loop(0, HQ_SHARD, load_head, 0)

        def compute_chunk(c):
            origin = lax.rem(my - c + 2 * N_DEV, N_DEV)
            x_c = xg[origin]
            q = lax.dot(x_c, wq_ref[:, :], preferred_element_type=jnp.float32)
            q_bf = q.astype(jnp.bfloat16)
            for h in range(HQ_SHARD):
                q_scr[h] = q_bf[:, h * DH:(h + 1) * DH]

            r = lax.broadcasted_iota(jnp.int32, (SQ_SHARD, SKV), 0)
            ccol = lax.broadcasted_iota(jnp.int32, (SQ_SHARD, SKV), 1)
            qb = r // BLK + origin * (SQ_SHARD // BLK)
            kb = ccol // BLK
            mask = (qb == kb) | (kb == 0) | (lax.rem(qb + kb, 3) == 0)

            def head_body(h, acc):
                s = lax.dot_general(
                    q_scr[h], k_bf[h], (((1,), (1,)), ((), ())),
                    preferred_element_type=jnp.float32) * SCALE
                s = jnp.where(mask, s, NEG_INF)
                mx = jnp.max(s, axis=1, keepdims=True)
                w = jnp.exp(s - mx)
                w = (w / jnp.sum(w, axis=1, keepdims=True)).astype(jnp.bfloat16)
                ctx = lax.dot_general(
                    w, v_bf[h], (((1,), (0,)), ((), ())),
                    preferred_element_type=jnp.float32)
                return acc + lax.dot(ctx.astype(jnp.bfloat16), wo_ref[h],
                                     preferred_element_type=jnp.float32)

            acc0 = jnp.zeros((SQ_SHARD, D_MODEL), jnp.float32)
            pbuf[origin] = lax.fori_loop(
                0, HQ_SHARD, head_body, acc0).astype(jnp.bfloat16)

        def make_ag(hop):
            slot = lax.rem(my - hop + 2 * N_DEV, N_DEV)
            return pltpu.make_async_remote_copy(
                src_ref=xg.at[slot], dst_ref=xg.at[slot],
                send_sem=ag_send.at[hop], recv_sem=ag_recv.at[hop],
                device_id=(right,), device_id_type=pl.DeviceIdType.MESH)

        def make_rs(s):
            c_send = lax.rem(my - 1 - s + 2 * N_DEV, N_DEV)
            return pltpu.make_async_remote_copy(
                src_ref=pbuf.at[c_send], dst_ref=rs_buf.at[s],
                send_sem=rs_send.at[s], recv_sem=rs_recv.at[s],
                device_id=(right,), device_id_type=pl.DeviceIdType.MESH)

        compute_chunk(0)
        ag0.wait()

        ag1 = make_ag(1)
        ag1.start()
        compute_chunk(1)
        ag1.wait()

        ag2 = make_ag(2)
        ag2.start()
        rs0 = make_rs(0)
        rs0.start()
        compute_chunk(2)
        ag2.wait()
        rs0.wait()
        c = lax.rem(my - 2 + 2 * N_DEV, N_DEV)
        pbuf[c] = pbuf[c] + rs_buf[0]

        rs1 = make_rs(1)
        rs1.start()
        compute_chunk(3)
        rs1.wait()
        c = lax.rem(my - 3 + 2 * N_DEV, N_DEV)
        pbuf[c] = pbuf[c] + rs_buf[1]

        rs2 = make_rs(2)
        rs2.start()
        rs2.wait()
        out_ref[0] = (pbuf[my] + rs_buf[2]).astype(jnp.float32)

    return pl.pallas_call(
        body,
        out_shape=jax.ShapeDtypeStruct((1, SQ_SHARD, D_MODEL), jnp.float32),
        in_specs=[
            pl.BlockSpec(memory_space=pltpu.VMEM),
            pl.BlockSpec(memory_space=pltpu.VMEM),
            pl.BlockSpec(memory_space=pl.ANY),
            pl.BlockSpec(memory_space=pl.ANY),
            pl.BlockSpec(memory_space=pltpu.VMEM),
        ],
        out_specs=pl.BlockSpec(memory_space=pltpu.VMEM),
        scratch_shapes=[
            pltpu.VMEM((N_DEV, SQ_SHARD, D_MODEL), jnp.bfloat16),
            pltpu.VMEM((N_DEV, SQ_SHARD, D_MODEL), jnp.bfloat16),
            pltpu.VMEM((N_DEV - 1, SQ_SHARD, D_MODEL), jnp.bfloat16),
            pltpu.VMEM((HQ_SHARD, SKV, DH), jnp.bfloat16),
            pltpu.VMEM((HQ_SHARD, SKV, DH), jnp.bfloat16),
            pltpu.VMEM((SKV, DH), jnp.float32),
            pltpu.VMEM((HQ_SHARD, SQ_SHARD, DH), jnp.bfloat16),
            pltpu.SemaphoreType.DMA((1,)),
            pltpu.SemaphoreType.DMA((3,)),
            pltpu.SemaphoreType.DMA((3,)),
            pltpu.SemaphoreType.DMA((3,)),
            pltpu.SemaphoreType.DMA((3,)),
        ],
        compiler_params=pltpu.CompilerParams(
            collective_id=0, vmem_limit_bytes=60 * 1024 * 1024),
    )(x.astype(jnp.bfloat16), Wq.astype(jnp.bfloat16), K_ext, V_ext,
      Wo.astype(jnp.bfloat16).reshape(HQ_SHARD, DH, D_MODEL))


# device time: 131681 ns/iter; 1.3197x vs baseline; 1.1341x over previous
import jax
import jax.numpy as jnp
from jax import lax
from jax.experimental import pallas as pl
from jax.experimental.pallas import tpu as pltpu

N_DEV = 4
SQ_SHARD = 256
HQ_SHARD = 8
DH = 128
SKV = 4096
D_MODEL = 1024
BLK = 64
SCALE = 0.08838834764831843
NEG_INF = -1e9


def kernel(x, Wq, K_ext, V_ext, Wo):
    def body(x_ref, wq_ref, k_hbm, v_hbm, wo_ref, out_ref,
             xg, pbuf, rs_buf, k_bf, v_bf, kv_stage, q_scr,
             kv_sem, ag_send, ag_recv, rs_send, rs_recv):
        my = lax.axis_index("i")
        right = lax.rem(my + 1, N_DEV)
        left = lax.rem(my + N_DEV - 1, N_DEV)

        barrier = pltpu.get_barrier_semaphore()
        for nbr in (left, right):
            pl.semaphore_signal(barrier, inc=1, device_id=(nbr,),
                                device_id_type=pl.DeviceIdType.MESH)
        pl.semaphore_wait(barrier, 2)

        xg[my] = x_ref[0]
        ag0 = pltpu.make_async_remote_copy(
            src_ref=xg.at[my], dst_ref=xg.at[my],
            send_sem=ag_send.at[0], recv_sem=ag_recv.at[0],
            device_id=(right,), device_id_type=pl.DeviceIdType.MESH)
        ag0.start()

        def make_load(hbm, h, slot):
            return pltpu.make_async_copy(
                hbm.at[0, :, my * HQ_SHARD + h, :], kv_stage.at[slot],
                kv_sem.at[slot])

        make_load(k_hbm, 0, 0).start()
        make_load(v_hbm, 0, 2).start()
        for h in range(HQ_SHARD):
            if h + 1 < HQ_SHARD:
                make_load(k_hbm, h + 1, (h + 1) % 2).start()
                make_load(v_hbm, h + 1, 2 + (h + 1) % 2).start()
            make_load(k_hbm, h, h % 2).wait()
            k_bf[h] = kv_stage[h % 2].astype(jnp.bfloat16)
            make_load(v_hbm, h, 2 + h % 2).wait()
            v_bf[h] = kv_stage[2 + h % 2].astype(jnp.bfloat16)

        def compute_chunk(c):
            origin = lax.rem(my - c + 2 * N_DEV, N_DEV)
            x_c = xg[origin]
            q = lax.dot(x_c, wq_ref[:, :], preferred_element_type=jnp.float32)
            q_bf = q.astype(jnp.bfloat16)
            for h in range(HQ_SHARD):
                q_scr[h] = q_bf[:, h * DH:(h + 1) * DH]

            r = lax.broadcasted_iota(jnp.int32, (SQ_SHARD, SKV), 0)
            ccol = lax.broadcasted_iota(jnp.int32, (SQ_SHARD, SKV), 1)
            qb = r // BLK + origin * (SQ_SHARD // BLK)
            kb = ccol // BLK
            mask = (qb == kb) | (kb == 0) | (lax.rem(qb + kb, 3) == 0)

            def head_body(h, acc):
                s = lax.dot_general(
                    q_scr[h], k_bf[h], (((1,), (1,)), ((), ())),
                    preferred_element_type=jnp.float32) * SCALE
                s = jnp.where(mask, s, NEG_INF)
                mx = jnp.max(s, axis=1, keepdims=True)
                w = jnp.exp(s - mx)
                w = (w / jnp.sum(w, axis=1, keepdims=True)).astype(jnp.bfloat16)
                ctx = lax.dot_general(
                    w, v_bf[h], (((1,), (0,)), ((), ())),
                    preferred_element_type=jnp.float32)
                return acc + lax.dot(ctx.astype(jnp.bfloat16), wo_ref[h],
                                     preferred_element_type=jnp.float32)

            acc0 = jnp.zeros((SQ_SHARD, D_MODEL), jnp.float32)
            pbuf[origin] = lax.fori_loop(
                0, HQ_SHARD, head_body, acc0).astype(jnp.bfloat16)

        def make_ag(hop):
            slot = lax.rem(my - hop + 2 * N_DEV, N_DEV)
            return pltpu.make_async_remote_copy(
                src_ref=xg.at[slot], dst_ref=xg.at[slot],
                send_sem=ag_send.at[hop], recv_sem=ag_recv.at[hop],
                device_id=(right,), device_id_type=pl.DeviceIdType.MESH)

        def make_rs(s):
            c_send = lax.rem(my - 1 - s + 2 * N_DEV, N_DEV)
            return pltpu.make_async_remote_copy(
                src_ref=pbuf.at[c_send], dst_ref=rs_buf.at[s],
                send_sem=rs_send.at[s], recv_sem=rs_recv.at[s],
                device_id=(right,), device_id_type=pl.DeviceIdType.MESH)

        compute_chunk(0)
        ag0.wait()

        ag1 = make_ag(1)
        ag1.start()
        compute_chunk(1)
        ag1.wait()

        ag2 = make_ag(2)
        ag2.start()
        rs0 = make_rs(0)
        rs0.start()
        compute_chunk(2)
        ag2.wait()
        rs0.wait()
        c = lax.rem(my - 2 + 2 * N_DEV, N_DEV)
        pbuf[c] = pbuf[c] + rs_buf[0]

        rs1 = make_rs(1)
        rs1.start()
        compute_chunk(3)
        rs1.wait()
        c = lax.rem(my - 3 + 2 * N_DEV, N_DEV)
        pbuf[c] = pbuf[c] + rs_buf[1]

        rs2 = make_rs(2)
        rs2.start()
        rs2.wait()
        out_ref[0] = (pbuf[my] + rs_buf[2]).astype(jnp.float32)

    return pl.pallas_call(
        body,
        out_shape=jax.ShapeDtypeStruct((1, SQ_SHARD, D_MODEL), jnp.float32),
        in_specs=[
            pl.BlockSpec(memory_space=pltpu.VMEM),
            pl.BlockSpec(memory_space=pltpu.VMEM),
            pl.BlockSpec(memory_space=pl.ANY),
            pl.BlockSpec(memory_space=pl.ANY),
            pl.BlockSpec(memory_space=pltpu.VMEM),
        ],
        out_specs=pl.BlockSpec(memory_space=pltpu.VMEM),
        scratch_shapes=[
            pltpu.VMEM((N_DEV, SQ_SHARD, D_MODEL), jnp.bfloat16),
            pltpu.VMEM((N_DEV, SQ_SHARD, D_MODEL), jnp.bfloat16),
            pltpu.VMEM((N_DEV - 1, SQ_SHARD, D_MODEL), jnp.bfloat16),
            pltpu.VMEM((HQ_SHARD, SKV, DH), jnp.bfloat16),
            pltpu.VMEM((HQ_SHARD, SKV, DH), jnp.bfloat16),
            pltpu.VMEM((4, SKV, DH), jnp.float32),
            pltpu.VMEM((HQ_SHARD, SQ_SHARD, DH), jnp.bfloat16),
            pltpu.SemaphoreType.DMA((4,)),
            pltpu.SemaphoreType.DMA((3,)),
            pltpu.SemaphoreType.DMA((3,)),
            pltpu.SemaphoreType.DMA((3,)),
            pltpu.SemaphoreType.DMA((3,)),
        ],
        compiler_params=pltpu.CompilerParams(
            collective_id=0, vmem_limit_bytes=60 * 1024 * 1024),
    )(x.astype(jnp.bfloat16), Wq.astype(jnp.bfloat16), K_ext, V_ext,
      Wo.astype(jnp.bfloat16).reshape(HQ_SHARD, DH, D_MODEL))
